# baseline (speedup 1.0000x reference)
import functools

import jax
import jax.numpy as jnp
from jax import lax
from jax.experimental import pallas as pl
from jax.experimental.pallas import tpu as pltpu
from jax.experimental.pallas import tpu_sc as plsc

DIM = 64
K = 10
LANES = 16
CHUNK = 64
NW = 32


def _sc_dots(cidx, pidx, nidx, ctab, xtab, b):
    bpw = b // NW
    n_chunks = bpw // CHUNK
    crows_pw = bpw // CHUNK
    nrows_pw = bpw * K // 128
    nrows_pc = CHUNK * K // 128
    mesh = plsc.VectorSubcoreMesh(core_axis_name="c", subcore_axis_name="s")
    nc = 2

    @functools.partial(
        pl.kernel,
        out_type=[
            jax.ShapeDtypeStruct((b,), jnp.float32),
            jax.ShapeDtypeStruct((b,), jnp.float32),
        ],
        mesh=mesh,
        compiler_params=pltpu.CompilerParams(needs_layout_passes=False),
        scratch_types=[
            pltpu.VMEM((crows_pw, CHUNK), jnp.int32),
            pltpu.VMEM((crows_pw, CHUNK), jnp.int32),
            pltpu.VMEM((nrows_pw, 128), jnp.int32),
            pltpu.VMEM((CHUNK, 128), jnp.float32),
            pltpu.VMEM((CHUNK, 128), jnp.float32),
            pltpu.VMEM((CHUNK * K, 128), jnp.float32),
            pltpu.VMEM((CHUNK,), jnp.float32),
            pltpu.VMEM((CHUNK,), jnp.float32),
            pltpu.SemaphoreType.DMA,
            pltpu.SemaphoreType.DMA,
            pltpu.SemaphoreType.DMA,
            pltpu.SemaphoreType.DMA,
            pltpu.SemaphoreType.DMA,
            pltpu.SemaphoreType.DMA,
        ],
    )
    def sc_kern(cidx_hbm, pidx_hbm, nidx_hbm, ctab_hbm, xtab_hbm,
                pos_out, neg_out,
                cidx_v, pidx_v, nidx_v, crow, prow, nrow, posb, negb, sem,
                nsem0, nsem1, nsem2, nsem3, nsem4):
        nsems = [nsem0, nsem1, nsem2, nsem3, nsem4]
        wid = lax.axis_index("s") * nc + lax.axis_index("c")
        pltpu.sync_copy(cidx_hbm.at[pl.ds(wid * crows_pw, crows_pw)], cidx_v)
        pltpu.sync_copy(pidx_hbm.at[pl.ds(wid * crows_pw, crows_pw)], pidx_v)
        pltpu.sync_copy(nidx_hbm.at[pl.ds(wid * nrows_pw, nrows_pw)], nidx_v)
        iota = lax.iota(jnp.int32, LANES)

        for m in range(n_chunks):
            copies = [
                pltpu.async_copy(ctab_hbm.at[cidx_v.at[m]], crow, sem),
                pltpu.async_copy(xtab_hbm.at[pidx_v.at[m]], prow, sem),
            ]
            ncopies = [
                pltpu.async_copy(
                    xtab_hbm.at[nidx_v.at[m * nrows_pc + j]],
                    nrow.at[pl.ds(j * 128, 128)], nsems[j])
                for j in range(nrows_pc)]
            for c in copies:
                c.wait()

            waited = 0
            for g in range(CHUNK // LANES):
                need = -(-((g + 1) * LANES * K) // 128)
                while waited < need:
                    ncopies[waited].wait()
                    waited += 1
                ev = g * LANES + iota
                nrows = [ev * K + k for k in range(K)]

                def dstep(d, acc):
                    pacc, nacc = acc
                    dv = jnp.zeros((LANES,), jnp.int32) + d
                    cd = plsc.load_gather(crow, [ev, dv])
                    pd_ = plsc.load_gather(prow, [ev, dv])
                    nsd = plsc.load_gather(nrow, [nrows[0], dv])
                    for k in range(1, K):
                        nsd = nsd + plsc.load_gather(nrow, [nrows[k], dv])
                    return pacc + cd * pd_, nacc + cd * nsd

                z = jnp.zeros((LANES,), jnp.float32)
                pacc, nacc = lax.fori_loop(0, DIM, dstep, (z, z), unroll=4)
                posb[pl.ds(g * LANES, LANES)] = pacc
                negb[pl.ds(g * LANES, LANES)] = nacc

            base = wid * bpw + m * CHUNK
            pltpu.sync_copy(posb, pos_out.at[pl.ds(base, CHUNK)])
            pltpu.sync_copy(negb, neg_out.at[pl.ds(base, CHUNK)])

    return sc_kern(cidx, pidx, nidx, ctab, xtab)


def _log_sigmoid(x):
    return jnp.minimum(x, 0.0) - jnp.log1p(jnp.exp(-jnp.abs(x)))


def _tc_reduce_body(pos_ref, neg_ref, out_ref):
    tot = (jnp.sum(_log_sigmoid(pos_ref[...]))
           + jnp.sum(_log_sigmoid(-neg_ref[...])))
    out_ref[0, 0] = tot


def kernel(center, positive_context, negative_context, batch_size,
           center_table, context_table):
    b = center.shape[0]
    cidx = center.astype(jnp.int32).reshape(b // CHUNK, CHUNK)
    pidx = positive_context.astype(jnp.int32).reshape(b // CHUNK, CHUNK)
    nidx = negative_context.astype(jnp.int32).reshape(b * K // 128, 128)
    ctab = jnp.pad(center_table, ((0, 0), (0, 128 - DIM)))
    xtab = jnp.pad(context_table, ((0, 0), (0, 128 - DIM)))

    pos_dot, neg_dot = _sc_dots(cidx, pidx, nidx, ctab, xtab, b)

    tot = pl.pallas_call(
        _tc_reduce_body,
        out_shape=jax.ShapeDtypeStruct((1, 1), jnp.float32),
        out_specs=pl.BlockSpec(memory_space=pltpu.SMEM),
    )(pos_dot.reshape(b // 128, 128), neg_dot.reshape(b // 128, 128))
    return -tot[0, 0] / batch_size
